# Initial kernel scaffold; baseline (speedup 1.0000x reference)
import jax
import jax.numpy as jnp
from jax.experimental import pallas as pl


def kernel(sigma_indices, u, axis, cdf, omega_grid):
    raise NotImplementedError("write your pallas kernel here")


# speedup vs baseline: 30.7505x; 30.7505x over previous
import functools

import jax
import jax.numpy as jnp
from jax import lax
from jax.experimental import pallas as pl
from jax.experimental.pallas import tpu as pltpu
from jax.experimental.pallas import tpu_sc as plsc

_NUM_OMEGA = 1000
_NUM_SIGMA = 32
_N = 65536
_TOL = 1e-7

_NC = 2
_NS = 16
_L = 16
_NW = _NC * _NS
_CHUNK = _N // _NW
_NVEC = _CHUNK // _L

_mesh = plsc.VectorSubcoreMesh(core_axis_name="c", subcore_axis_name="s")


@functools.partial(
    pl.kernel,
    out_type=jax.ShapeDtypeStruct((_N,), jnp.float32),
    mesh=_mesh,
    compiler_params=pltpu.CompilerParams(needs_layout_passes=False),
    scratch_types=[
        pltpu.VMEM((_NUM_SIGMA * _NUM_OMEGA,), jnp.float32),
        pltpu.VMEM((_NUM_OMEGA,), jnp.float32),
        pltpu.VMEM((_CHUNK,), jnp.int32),
        pltpu.VMEM((_CHUNK,), jnp.float32),
        pltpu.VMEM((_CHUNK,), jnp.float32),
    ],
)
def _sc_sample(sig_hbm, u_hbm, cdf_hbm, og_hbm, om_hbm,
               cdf_v, og_v, sig_v, u_v, om_v):
    wid = lax.axis_index("s") * _NC + lax.axis_index("c")
    base = wid * _CHUNK
    pltpu.sync_copy(cdf_hbm, cdf_v)
    pltpu.sync_copy(og_hbm, og_v)
    pltpu.sync_copy(sig_hbm.at[pl.ds(base, _CHUNK)], sig_v)
    pltpu.sync_copy(u_hbm.at[pl.ds(base, _CHUNK)], u_v)

    def body(i, carry):
        off = pl.multiple_of(i * _L, _L)
        s = sig_v[pl.ds(off, _L)]
        uu = u_v[pl.ds(off, _L)]
        rb = s * _NUM_OMEGA
        pos = jnp.zeros((_L,), jnp.int32)
        for step in (512, 256, 128, 64, 32, 16, 8, 4, 2, 1):
            cand = pos + step
            candc = jnp.minimum(cand, _NUM_OMEGA - 1)
            val = plsc.load_gather(cdf_v, [rb + candc])
            take = jnp.logical_and(cand <= _NUM_OMEGA - 1, val < uu)
            pos = jnp.where(take, cand, pos)
        idx = pos + 1
        c_lo = plsc.load_gather(cdf_v, [rb + pos])
        c_hi = plsc.load_gather(cdf_v, [rb + idx])
        o_lo = plsc.load_gather(og_v, [pos])
        o_hi = plsc.load_gather(og_v, [idx])
        denom = jnp.maximum(c_hi - c_lo, 1e-10)
        om_v[pl.ds(off, _L)] = o_lo + (uu - c_lo) * (o_hi - o_lo) / denom
        return carry

    lax.fori_loop(0, _NVEC, body, 0)
    pltpu.sync_copy(om_v, om_hbm.at[pl.ds(base, _CHUNK)])


_NB = 2048


def _tc_body(om_ref, ax_ref, out_ref):
    om = om_ref[...]
    x = ax_ref[0:1, :]
    y = ax_ref[1:2, :]
    z = ax_ref[2:3, :]
    nrm = jnp.sqrt(x * x + y * y + z * z)
    scale = om / (nrm + _TOL)
    rx = x * scale
    ry = y * scale
    rz = z * scale
    a = jnp.sqrt(rx * rx + ry * ry + rz * rz)
    mask = jnp.abs(a) < _TOL
    a_safe = jnp.where(mask, 1.0, a)
    sin_c = jnp.where(mask, 1.0 - a * a / 6.0, jnp.sin(a_safe) / a_safe)
    cos_c = jnp.where(mask, 0.5 - a * a / 24.0,
                      (1.0 - jnp.cos(a_safe)) / (a_safe * a_safe))
    xx = rx * rx
    yy = ry * ry
    zz = rz * rz
    xy = rx * ry
    xz = rx * rz
    yz = ry * rz
    r00 = 1.0 - cos_c * (yy + zz)
    r01 = cos_c * xy - sin_c * rz
    r02 = cos_c * xz + sin_c * ry
    r10 = cos_c * xy + sin_c * rz
    r11 = 1.0 - cos_c * (xx + zz)
    r12 = cos_c * yz - sin_c * rx
    r20 = cos_c * xz - sin_c * ry
    r21 = cos_c * yz + sin_c * rx
    r22 = 1.0 - cos_c * (xx + yy)
    out_ref[...] = jnp.concatenate(
        [r00, r01, r02, r10, r11, r12, r20, r21, r22], axis=0)


def _rodrigues(om2d, ax2d):
    return pl.pallas_call(
        _tc_body,
        grid=(_N // _NB,),
        in_specs=[
            pl.BlockSpec((1, _NB), lambda i: (0, i)),
            pl.BlockSpec((3, _NB), lambda i: (0, i)),
        ],
        out_specs=pl.BlockSpec((9, _NB), lambda i: (0, i)),
        out_shape=jax.ShapeDtypeStruct((9, _N), jnp.float32),
    )(om2d, ax2d)


def kernel(sigma_indices, u, axis, cdf, omega_grid):
    sig = sigma_indices.astype(jnp.int32)
    om = _sc_sample(sig, u.astype(jnp.float32),
                    cdf.reshape(-1).astype(jnp.float32),
                    omega_grid.astype(jnp.float32))
    r9 = _rodrigues(om.reshape(1, _N), axis.astype(jnp.float32).T)
    return r9.T.reshape(_N, 3, 3)


# speedup vs baseline: 37.0853x; 1.2060x over previous
import functools

import jax
import jax.numpy as jnp
from jax import lax
from jax.experimental import pallas as pl
from jax.experimental.pallas import tpu as pltpu
from jax.experimental.pallas import tpu_sc as plsc

_NUM_OMEGA = 1000
_NUM_SIGMA = 32
_N = 65536
_TOL = 1e-7

_NC = 2
_NS = 16
_L = 16
_NW = _NC * _NS
_CHUNK = _N // _NW
_NVEC = _CHUNK // _L

_mesh = plsc.VectorSubcoreMesh(core_axis_name="c", subcore_axis_name="s")


@functools.partial(
    pl.kernel,
    out_type=jax.ShapeDtypeStruct((_N,), jnp.float32),
    mesh=_mesh,
    compiler_params=pltpu.CompilerParams(needs_layout_passes=False),
    scratch_types=[
        pltpu.VMEM((_NUM_SIGMA * _NUM_OMEGA,), jnp.float32),
        pltpu.VMEM((_NUM_OMEGA,), jnp.float32),
        pltpu.VMEM((_CHUNK,), jnp.int32),
        pltpu.VMEM((_CHUNK,), jnp.float32),
        pltpu.VMEM((_CHUNK,), jnp.float32),
    ],
)
def _sc_sample(sig_hbm, u_hbm, cdf_hbm, og_hbm, om_hbm,
               cdf_v, og_v, sig_v, u_v, om_v):
    wid = lax.axis_index("s") * _NC + lax.axis_index("c")
    base = wid * _CHUNK
    pltpu.sync_copy(cdf_hbm, cdf_v)
    pltpu.sync_copy(og_hbm, og_v)
    pltpu.sync_copy(sig_hbm.at[pl.ds(base, _CHUNK)], sig_v)
    pltpu.sync_copy(u_hbm.at[pl.ds(base, _CHUNK)], u_v)

    @plsc.parallel_loop(0, _NVEC, unroll=8)
    def body(i):
        off = pl.multiple_of(i * _L, _L)
        s = sig_v[pl.ds(off, _L)]
        uu = u_v[pl.ds(off, _L)]
        rb = s * _NUM_OMEGA
        pos = jnp.zeros((_L,), jnp.int32)
        for step in (512, 256, 128, 64, 32, 16, 8, 4, 2, 1):
            cand = pos + step
            candc = jnp.minimum(cand, _NUM_OMEGA - 1)
            val = plsc.load_gather(cdf_v, [rb + candc])
            take = jnp.logical_and(cand <= _NUM_OMEGA - 1, val < uu)
            pos = jnp.where(take, cand, pos)
        idx = pos + 1
        c_lo = plsc.load_gather(cdf_v, [rb + pos])
        c_hi = plsc.load_gather(cdf_v, [rb + idx])
        o_lo = plsc.load_gather(og_v, [pos])
        o_hi = plsc.load_gather(og_v, [idx])
        denom = jnp.maximum(c_hi - c_lo, 1e-10)
        om_v[pl.ds(off, _L)] = o_lo + (uu - c_lo) * (o_hi - o_lo) / denom
    pltpu.sync_copy(om_v, om_hbm.at[pl.ds(base, _CHUNK)])


_NB = 2048


def _tc_body(om_ref, ax_ref, out_ref):
    om = om_ref[...]
    x = ax_ref[0:1, :]
    y = ax_ref[1:2, :]
    z = ax_ref[2:3, :]
    nrm = jnp.sqrt(x * x + y * y + z * z)
    scale = om / (nrm + _TOL)
    rx = x * scale
    ry = y * scale
    rz = z * scale
    a = jnp.sqrt(rx * rx + ry * ry + rz * rz)
    mask = jnp.abs(a) < _TOL
    a_safe = jnp.where(mask, 1.0, a)
    sin_c = jnp.where(mask, 1.0 - a * a / 6.0, jnp.sin(a_safe) / a_safe)
    cos_c = jnp.where(mask, 0.5 - a * a / 24.0,
                      (1.0 - jnp.cos(a_safe)) / (a_safe * a_safe))
    xx = rx * rx
    yy = ry * ry
    zz = rz * rz
    xy = rx * ry
    xz = rx * rz
    yz = ry * rz
    r00 = 1.0 - cos_c * (yy + zz)
    r01 = cos_c * xy - sin_c * rz
    r02 = cos_c * xz + sin_c * ry
    r10 = cos_c * xy + sin_c * rz
    r11 = 1.0 - cos_c * (xx + zz)
    r12 = cos_c * yz - sin_c * rx
    r20 = cos_c * xz - sin_c * ry
    r21 = cos_c * yz + sin_c * rx
    r22 = 1.0 - cos_c * (xx + yy)
    out_ref[...] = jnp.concatenate(
        [r00, r01, r02, r10, r11, r12, r20, r21, r22], axis=0)


def _rodrigues(om2d, ax2d):
    return pl.pallas_call(
        _tc_body,
        grid=(_N // _NB,),
        in_specs=[
            pl.BlockSpec((1, _NB), lambda i: (0, i)),
            pl.BlockSpec((3, _NB), lambda i: (0, i)),
        ],
        out_specs=pl.BlockSpec((9, _NB), lambda i: (0, i)),
        out_shape=jax.ShapeDtypeStruct((9, _N), jnp.float32),
    )(om2d, ax2d)


def kernel(sigma_indices, u, axis, cdf, omega_grid):
    sig = sigma_indices.astype(jnp.int32)
    om = _sc_sample(sig, u.astype(jnp.float32),
                    cdf.reshape(-1).astype(jnp.float32),
                    omega_grid.astype(jnp.float32))
    r9 = _rodrigues(om.reshape(1, _N), axis.astype(jnp.float32).T)
    return r9.T.reshape(_N, 3, 3)
